# Initial kernel scaffold; baseline (speedup 1.0000x reference)
import jax
import jax.numpy as jnp
from jax.experimental import pallas as pl


def kernel(token_sequence, svc_mask, svc_thresholds, svc_decay):
    raise NotImplementedError("write your pallas kernel here")


# speedup vs baseline: 1329.1828x; 1329.1828x over previous
import functools

import jax
import jax.numpy as jnp
from jax import lax
from jax.experimental import pallas as pl
from jax.experimental.pallas import tpu as pltpu
from jax.experimental.pallas import tpu_sc as plsc

_VOCAB = 100000
_SEQ = 8192
_L = 16
_NS = 16
_VP = 102400
_RPW = _VP // _NS
_NV = _RPW // _L
_TPW = _SEQ // _NS
_TR = _TPW // 128
_K = 5
_PAD_BASE = _VP - _L
_GAIN_UP = 1.5
_GAIN_DOWN = 0.6

_mesh = plsc.VectorSubcoreMesh(
    core_axis_name="c", subcore_axis_name="s", num_cores=2, num_subcores=_NS
)


def _lane_shuffle(v, perm):
    return v.at[perm].get(mode="promise_in_bounds")


def _lane_max_splat(v, lane):
    for sh in (8, 4, 2, 1):
        v = jnp.maximum(v, _lane_shuffle(v, lane ^ sh))
    return v


def _lane_sum_splat(v, lane):
    for sh in (8, 4, 2, 1):
        v = v + _lane_shuffle(v, lane ^ sh)
    return v


def _insert_top(tops, key):
    out = []
    for j, t in enumerate(tops):
        hi = jnp.maximum(t, key)
        if j + 1 < len(tops):
            key = jnp.minimum(t, key)
        out.append(hi)
    return out


@functools.partial(
    pl.kernel,
    out_type=(
        jax.ShapeDtypeStruct((_VP,), jnp.float32),
        jax.ShapeDtypeStruct((_L,), jnp.float32),
    ),
    mesh=_mesh,
    scratch_types=[
        pltpu.VMEM_SHARED((_VP,), jnp.float32),
        pltpu.VMEM_SHARED((_VP,), jnp.float32),
        pltpu.VMEM_SHARED((_VP,), jnp.float32),
        pltpu.VMEM_SHARED((_VP,), jnp.float32),
        pltpu.VMEM_SHARED((_NS * _K * _L,), jnp.int32),
        pltpu.VMEM_SHARED((_NS * _L,), jnp.float32),
        pltpu.VMEM((_TR, 128), jnp.int32),
        pltpu.VMEM((_TR, 128), jnp.float32),
        pltpu.VMEM((_RPW,), jnp.float32),
        pltpu.VMEM((_RPW,), jnp.float32),
        pltpu.VMEM((_RPW,), jnp.float32),
        pltpu.VMEM((_RPW,), jnp.float32),
        pltpu.VMEM((_RPW,), jnp.float32),
        pltpu.VMEM((_K * _L,), jnp.int32),
        pltpu.VMEM((_NS * _K * _L,), jnp.int32),
        pltpu.VMEM((_NS * _L,), jnp.float32),
        pltpu.VMEM((_L,), jnp.int32),
        pltpu.VMEM((_L,), jnp.float32),
        pltpu.VMEM((_L,), jnp.float32),
        pltpu.SemaphoreType.DMA,
    ],
)
def _spiking_sc_kernel(
    tok_hbm, m0_hbm, m1_hbm, m2_hbm, gains_hbm, svc_hbm,
    hist, hm0, hm1, hm2, topst, svcst,
    tok_v, val_v, cnt_v, m0_v, m1_v, m2_v, g_v,
    stage_v, allstage_v, allsvc_v, widx_v, wval_v, svco_v, sem,
):
    c = lax.axis_index("c")
    s = lax.axis_index("s")
    rbase = s * _RPW
    lane = lax.iota(jnp.int32, _L)
    zf = jnp.zeros((_L,), jnp.float32)
    zi = jnp.zeros((_L,), jnp.int32)

    def _zero_body(i, carry):
        cnt_v[pl.ds(pl.multiple_of(i * _L, _L), _L)] = zf
        return carry

    lax.fori_loop(0, _NV, _zero_body, 0)
    pltpu.sync_copy(cnt_v, hist.at[pl.ds(rbase, _RPW)])
    pltpu.sync_copy(cnt_v, hm0.at[pl.ds(rbase, _RPW)])
    pltpu.sync_copy(cnt_v, hm1.at[pl.ds(rbase, _RPW)])
    pltpu.sync_copy(cnt_v, hm2.at[pl.ds(rbase, _RPW)])
    plsc.subcore_barrier()

    pltpu.sync_copy(tok_hbm.at[pl.ds(s * _TR, _TR)], tok_v)
    ones = jnp.ones((_L,), jnp.float32)
    for j in range(_TR):
        for k in range(128 // _L):
            val_v[j, pl.ds(k * _L, _L)] = ones
    for j in range(_TR):
        pltpu.sync_copy(val_v.at[j], hist.at[tok_v.at[j]], add=True)
    for mh, msrc in ((hm0, m0_hbm), (hm1, m1_hbm), (hm2, m2_hbm)):
        pltpu.sync_copy(msrc.at[pl.ds(s * _TR, _TR)], val_v)
        for j in range(_TR):
            pltpu.sync_copy(val_v.at[j], mh.at[tok_v.at[j]], add=True)
    plsc.subcore_barrier()

    pltpu.sync_copy(hist.at[pl.ds(rbase, _RPW)], cnt_v)
    pltpu.sync_copy(hm0.at[pl.ds(rbase, _RPW)], m0_v)
    pltpu.sync_copy(hm1.at[pl.ds(rbase, _RPW)], m1_v)
    pltpu.sync_copy(hm2.at[pl.ds(rbase, _RPW)], m2_v)

    def _scan_body(i, carry):
        t1, t2, t3, t4, t5, a0, a1, a2 = carry
        off = pl.multiple_of(i * _L, _L)
        cnt = cnt_v[pl.ds(off, _L)]
        pos = cnt > 0.0
        g_v[pl.ds(off, _L)] = jnp.where(pos, _GAIN_DOWN, 1.0)
        a0 = a0 + cnt * m0_v[pl.ds(off, _L)]
        a1 = a1 + cnt * m1_v[pl.ds(off, _L)]
        a2 = a2 + cnt * m2_v[pl.ds(off, _L)]
        gidx = rbase + i * _L + lane
        key = jnp.where(pos, (cnt.astype(jnp.int32) << 17) | (0x1FFFF - gidx), 0)
        t1, t2, t3, t4, t5 = _insert_top([t1, t2, t3, t4, t5], key)
        return t1, t2, t3, t4, t5, a0, a1, a2

    t1, t2, t3, t4, t5, a0, a1, a2 = lax.fori_loop(
        0, _NV, _scan_body, (zi, zi, zi, zi, zi, zf, zf, zf)
    )

    @pl.when(c == 0)
    def _():
        pltpu.sync_copy(g_v, gains_hbm.at[pl.ds(rbase, _RPW)])

    for j, t in enumerate((t1, t2, t3, t4, t5)):
        stage_v[pl.ds(j * _L, _L)] = t
    pltpu.sync_copy(stage_v, topst.at[pl.ds(s * _K * _L, _K * _L)])
    sa0 = _lane_sum_splat(a0, lane)
    sa1 = _lane_sum_splat(a1, lane)
    sa2 = _lane_sum_splat(a2, lane)
    svco_v[...] = jnp.where(
        lane == 0, sa0, jnp.where(lane == 1, sa1, jnp.where(lane == 2, sa2, 0.0))
    )
    pltpu.sync_copy(svco_v, svcst.at[pl.ds(s * _L, _L)])
    plsc.subcore_barrier()

    @pl.when((c == 0) & (s == 0))
    def _():
        pltpu.sync_copy(topst, allstage_v)
        pltpu.sync_copy(svcst, allsvc_v)
        g = [allstage_v[pl.ds(j * _L, _L)] for j in range(_K)]
        for t in range(1, _NS):
            for j in range(_K):
                g = _insert_top(g, allstage_v[pl.ds(t * _K * _L + j * _L, _L)])
        widx = _PAD_BASE + lane
        prev = None
        for p in range(_K):
            if prev is None:
                cand = g[0]
            else:
                cand = zi
                for t in g:
                    cand = jnp.maximum(cand, jnp.where(t < prev, t, 0))
            m = _lane_max_splat(cand, lane)
            ok = m >= (1 << 17)
            idx = 0x1FFFF - (m & 0x1FFFF)
            widx = jnp.where((lane == p) & ok, idx, widx)
            prev = m
        widx_v[...] = widx
        wval_v[...] = jnp.full((_L,), _GAIN_UP, jnp.float32)
        pltpu.async_copy(wval_v, gains_hbm.at[widx_v], sem).wait()
        acc = zf
        for t in range(_NS):
            acc = acc + allsvc_v[pl.ds(t * _L, _L)]
        svco_v[...] = acc
        pltpu.sync_copy(svco_v, svc_hbm)


@jax.jit
def kernel(token_sequence, svc_mask, svc_thresholds, svc_decay):
    del svc_thresholds, svc_decay
    tok = token_sequence.astype(jnp.int32).reshape(_SEQ // 128, 128)
    mf = (svc_mask > 0).astype(jnp.float32)
    m0 = mf[:, 0].reshape(_SEQ // 128, 128)
    m1 = mf[:, 1].reshape(_SEQ // 128, 128)
    m2 = mf[:, 2].reshape(_SEQ // 128, 128)
    gains_pad, svc_pad = _spiking_sc_kernel(tok, m0, m1, m2)
    return gains_pad[:_VOCAB], svc_pad[:3]


# speedup vs baseline: 1432.2915x; 1.0776x over previous
import functools

import jax
import jax.numpy as jnp
from jax import lax
from jax.experimental import pallas as pl
from jax.experimental.pallas import tpu as pltpu
from jax.experimental.pallas import tpu_sc as plsc

_VOCAB = 100000
_SEQ = 8192
_L = 16
_NS = 16
_VP = 102400
_RPW = _VP // _NS
_NV = _RPW // _L
_TPW = _SEQ // _NS
_TR = _TPW // 128
_K = 5
_PAD_BASE = _VP - _L
_GAIN_UP = 1.5
_GAIN_DOWN = 0.6

_mesh = plsc.VectorSubcoreMesh(
    core_axis_name="c", subcore_axis_name="s", num_cores=2, num_subcores=_NS
)


def _lane_shuffle(v, perm):
    return v.at[perm].get(mode="promise_in_bounds")


def _lane_max_splat(v, lane):
    for sh in (8, 4, 2, 1):
        v = jnp.maximum(v, _lane_shuffle(v, lane ^ sh))
    return v


def _lane_sum_splat(v, lane):
    for sh in (8, 4, 2, 1):
        v = v + _lane_shuffle(v, lane ^ sh)
    return v


def _insert_top(tops, key):
    out = []
    for j, t in enumerate(tops):
        hi = jnp.maximum(t, key)
        if j + 1 < len(tops):
            key = jnp.minimum(t, key)
        out.append(hi)
    return out


@functools.partial(
    pl.kernel,
    out_type=(
        jax.ShapeDtypeStruct((_VP,), jnp.float32),
        jax.ShapeDtypeStruct((_L,), jnp.float32),
    ),
    mesh=_mesh,
    scratch_types=[
        pltpu.VMEM_SHARED((_VP,), jnp.float32),
        pltpu.VMEM_SHARED((_NS * _K * _L,), jnp.int32),
        pltpu.VMEM_SHARED((_NS * _L,), jnp.float32),
        pltpu.VMEM((_TR, 128), jnp.int32),
        pltpu.VMEM((_TR, 128), jnp.float32),
        pltpu.VMEM((_TR, 128), jnp.float32),
        pltpu.VMEM((_TR, 128), jnp.float32),
        pltpu.VMEM((_TR, 128), jnp.float32),
        pltpu.VMEM((_TR, 128), jnp.float32),
        pltpu.VMEM((_RPW,), jnp.float32),
        pltpu.VMEM((_RPW,), jnp.float32),
        pltpu.VMEM((_K * _L,), jnp.int32),
        pltpu.VMEM((_NS * _K * _L,), jnp.int32),
        pltpu.VMEM((_NS * _L,), jnp.float32),
        pltpu.VMEM((_L,), jnp.int32),
        pltpu.VMEM((_L,), jnp.float32),
        pltpu.VMEM((_L,), jnp.float32),
        pltpu.SemaphoreType.DMA,
    ],
)
def _spiking_sc_kernel(
    tok_hbm, m0_hbm, m1_hbm, m2_hbm, gains_hbm, svc_hbm,
    hist, topst, svcst,
    tok_v, val_v, cntg_v, mk0_v, mk1_v, mk2_v, cnt_v, g_v,
    stage_v, allstage_v, allsvc_v, widx_v, wval_v, svco_v, sem,
):
    c = lax.axis_index("c")
    s = lax.axis_index("s")
    rbase = s * _RPW
    lane = lax.iota(jnp.int32, _L)
    zf = jnp.zeros((_L,), jnp.float32)
    zi = jnp.zeros((_L,), jnp.int32)

    def _zero_body(i, carry):
        cnt_v[pl.ds(pl.multiple_of(i * _L, _L), _L)] = zf
        return carry

    lax.fori_loop(0, _NV, _zero_body, 0)
    pltpu.sync_copy(cnt_v, hist.at[pl.ds(rbase, _RPW)])
    plsc.subcore_barrier()

    pltpu.sync_copy(tok_hbm.at[pl.ds(s * _TR, _TR)], tok_v)
    ones = jnp.ones((_L,), jnp.float32)
    for j in range(_TR):
        for k in range(128 // _L):
            val_v[j, pl.ds(k * _L, _L)] = ones
    for j in range(_TR):
        pltpu.sync_copy(val_v.at[j], hist.at[tok_v.at[j]], add=True)
    plsc.subcore_barrier()

    pltpu.sync_copy(hist.at[pl.ds(rbase, _RPW)], cnt_v)
    for j in range(_TR):
        pltpu.sync_copy(hist.at[tok_v.at[j]], cntg_v.at[j])
    for msrc, mdst in ((m0_hbm, mk0_v), (m1_hbm, mk1_v), (m2_hbm, mk2_v)):
        pltpu.sync_copy(msrc.at[pl.ds(s * _TR, _TR)], mdst)

    def _scan_body(i, carry):
        t1, t2, t3, t4, t5, kidx = carry
        off = pl.multiple_of(i * _L, _L)
        cnt = cnt_v[pl.ds(off, _L)]
        pos = cnt > 0.0
        g_v[pl.ds(off, _L)] = jnp.where(pos, _GAIN_DOWN, 1.0)
        key = jnp.where(pos, (cnt.astype(jnp.int32) << 17) | kidx, 0)
        t1, t2, t3, t4, t5 = _insert_top([t1, t2, t3, t4, t5], key)
        return t1, t2, t3, t4, t5, kidx - _L

    kidx0 = (0x1FFFF - rbase) - lane
    t1, t2, t3, t4, t5, _ = lax.fori_loop(
        0, _NV, _scan_body, (zi, zi, zi, zi, zi, kidx0)
    )

    @pl.when(c == 0)
    def _():
        pltpu.sync_copy(g_v, gains_hbm.at[pl.ds(rbase, _RPW)])

    for j, t in enumerate((t1, t2, t3, t4, t5)):
        stage_v[pl.ds(j * _L, _L)] = t
    pltpu.sync_copy(stage_v, topst.at[pl.ds(s * _K * _L, _K * _L)])

    a0 = zf
    a1 = zf
    a2 = zf
    for j in range(_TR):
        for k in range(128 // _L):
            cg = cntg_v[j, pl.ds(k * _L, _L)]
            a0 = a0 + cg * mk0_v[j, pl.ds(k * _L, _L)]
            a1 = a1 + cg * mk1_v[j, pl.ds(k * _L, _L)]
            a2 = a2 + cg * mk2_v[j, pl.ds(k * _L, _L)]
    sa0 = _lane_sum_splat(a0, lane)
    sa1 = _lane_sum_splat(a1, lane)
    sa2 = _lane_sum_splat(a2, lane)
    svco_v[...] = jnp.where(
        lane == 0, sa0, jnp.where(lane == 1, sa1, jnp.where(lane == 2, sa2, 0.0))
    )
    pltpu.sync_copy(svco_v, svcst.at[pl.ds(s * _L, _L)])
    plsc.subcore_barrier()

    @pl.when((c == 0) & (s == 0))
    def _():
        pltpu.sync_copy(topst, allstage_v)
        pltpu.sync_copy(svcst, allsvc_v)
        g = [allstage_v[pl.ds(j * _L, _L)] for j in range(_K)]
        for t in range(1, _NS):
            for j in range(_K):
                g = _insert_top(g, allstage_v[pl.ds(t * _K * _L + j * _L, _L)])
        widx = _PAD_BASE + lane
        prev = None
        for p in range(_K):
            if prev is None:
                cand = g[0]
            else:
                cand = zi
                for t in g:
                    cand = jnp.maximum(cand, jnp.where(t < prev, t, 0))
            m = _lane_max_splat(cand, lane)
            ok = m >= (1 << 17)
            idx = 0x1FFFF - (m & 0x1FFFF)
            widx = jnp.where((lane == p) & ok, idx, widx)
            prev = m
        widx_v[...] = widx
        wval_v[...] = jnp.full((_L,), _GAIN_UP, jnp.float32)
        pltpu.async_copy(wval_v, gains_hbm.at[widx_v], sem).wait()
        acc = zf
        for t in range(_NS):
            acc = acc + allsvc_v[pl.ds(t * _L, _L)]
        svco_v[...] = acc
        pltpu.sync_copy(svco_v, svc_hbm)


@jax.jit
def kernel(token_sequence, svc_mask, svc_thresholds, svc_decay):
    del svc_thresholds, svc_decay
    tok = token_sequence.astype(jnp.int32).reshape(_SEQ // 128, 128)
    mf = (svc_mask > 0).astype(jnp.float32)
    m0 = mf[:, 0].reshape(_SEQ // 128, 128)
    m1 = mf[:, 1].reshape(_SEQ // 128, 128)
    m2 = mf[:, 2].reshape(_SEQ // 128, 128)
    gains_pad, svc_pad = _spiking_sc_kernel(tok, m0, m1, m2)
    return gains_pad[:_VOCAB], svc_pad[:3]


# speedup vs baseline: 1498.7155x; 1.0464x over previous
import functools

import jax
import jax.numpy as jnp
from jax import lax
from jax.experimental import pallas as pl
from jax.experimental.pallas import tpu as pltpu
from jax.experimental.pallas import tpu_sc as plsc

_VOCAB = 100000
_SEQ = 8192
_L = 16
_NS = 16
_VP = 102400
_RPW = _VP // _NS
_NV = _RPW // _L
_TPW = _SEQ // _NS
_TR = _TPW // 128
_K = 5
_PAD_BASE = _VP - _L
_GAIN_UP = 1.5
_GAIN_DOWN = 0.6

_mesh = plsc.VectorSubcoreMesh(
    core_axis_name="c", subcore_axis_name="s", num_cores=2, num_subcores=_NS
)


def _lane_shuffle(v, perm):
    return v.at[perm].get(mode="promise_in_bounds")


def _lane_max_splat(v, lane):
    for sh in (8, 4, 2, 1):
        v = jnp.maximum(v, _lane_shuffle(v, lane ^ sh))
    return v


def _lane_sum_splat(v, lane):
    for sh in (8, 4, 2, 1):
        v = v + _lane_shuffle(v, lane ^ sh)
    return v


def _insert_top(tops, key):
    out = []
    for j, t in enumerate(tops):
        hi = jnp.maximum(t, key)
        if j + 1 < len(tops):
            key = jnp.minimum(t, key)
        out.append(hi)
    return out


@functools.partial(
    pl.kernel,
    out_type=(
        jax.ShapeDtypeStruct((_VP,), jnp.float32),
        jax.ShapeDtypeStruct((_L,), jnp.float32),
    ),
    mesh=_mesh,
    scratch_types=[
        pltpu.VMEM_SHARED((_VP,), jnp.float32),
        pltpu.VMEM_SHARED((_NS * _K * _L,), jnp.int32),
        pltpu.VMEM_SHARED((_NS * _L,), jnp.float32),
        pltpu.VMEM((_TR, 128), jnp.int32),
        pltpu.VMEM((_TR, 128), jnp.float32),
        pltpu.VMEM((_TR, 128), jnp.float32),
        pltpu.VMEM((_TR, 128), jnp.float32),
        pltpu.VMEM((_TR, 128), jnp.float32),
        pltpu.VMEM((_TR, 128), jnp.float32),
        pltpu.VMEM((_RPW,), jnp.float32),
        pltpu.VMEM((_RPW,), jnp.float32),
        pltpu.VMEM((_K * _L,), jnp.int32),
        pltpu.VMEM((_NS * _K * _L,), jnp.int32),
        pltpu.VMEM((_NS * _L,), jnp.float32),
        pltpu.VMEM((_L,), jnp.int32),
        pltpu.VMEM((_L,), jnp.float32),
        pltpu.VMEM((_L,), jnp.float32),
        pltpu.SemaphoreType.DMA,
    ],
)
def _spiking_sc_kernel(
    tok_hbm, m0_hbm, m1_hbm, m2_hbm, gains_hbm, svc_hbm,
    hist, topst, svcst,
    tok_v, val_v, cntg_v, mk0_v, mk1_v, mk2_v, cnt_v, g_v,
    stage_v, allstage_v, allsvc_v, widx_v, wval_v, svco_v, sem,
):
    c = lax.axis_index("c")
    s = lax.axis_index("s")
    rbase = s * _RPW
    lane = lax.iota(jnp.int32, _L)
    zf = jnp.zeros((_L,), jnp.float32)
    zi = jnp.zeros((_L,), jnp.int32)

    def _zero_body(i, carry):
        for u in range(4):
            cnt_v[pl.ds(pl.multiple_of(i * (4 * _L) + u * _L, _L), _L)] = zf
        return carry

    lax.fori_loop(0, _NV // 4, _zero_body, 0)
    pltpu.sync_copy(cnt_v, hist.at[pl.ds(rbase, _RPW)])
    plsc.subcore_barrier()

    pltpu.sync_copy(tok_hbm.at[pl.ds(s * _TR, _TR)], tok_v)
    ones = jnp.ones((_L,), jnp.float32)
    for j in range(_TR):
        for k in range(128 // _L):
            val_v[j, pl.ds(k * _L, _L)] = ones
    for j in range(_TR):
        pltpu.sync_copy(val_v.at[j], hist.at[tok_v.at[j]], add=True)
    plsc.subcore_barrier()

    pltpu.sync_copy(hist.at[pl.ds(rbase, _RPW)], cnt_v)
    for j in range(_TR):
        pltpu.sync_copy(hist.at[tok_v.at[j]], cntg_v.at[j])
    for msrc, mdst in ((m0_hbm, mk0_v), (m1_hbm, mk1_v), (m2_hbm, mk2_v)):
        pltpu.sync_copy(msrc.at[pl.ds(s * _TR, _TR)], mdst)

    def _scan_body(i, carry):
        t1, t2, t3, t4, t5, kidx = carry
        tops = [t1, t2, t3, t4, t5]
        for u in range(4):
            off = pl.multiple_of(i * (4 * _L) + u * _L, _L)
            cnt = cnt_v[pl.ds(off, _L)]
            g_v[pl.ds(off, _L)] = jnp.where(cnt > 0.0, _GAIN_DOWN, 1.0)
            key = (cnt.astype(jnp.int32) << 17) | (kidx - u * _L)
            tops = _insert_top(tops, key)
        t1, t2, t3, t4, t5 = tops
        return t1, t2, t3, t4, t5, kidx - 4 * _L

    kidx0 = (0x1FFFF - rbase) - lane
    t1, t2, t3, t4, t5, _ = lax.fori_loop(
        0, _NV // 4, _scan_body, (zi, zi, zi, zi, zi, kidx0)
    )

    @pl.when(c == 0)
    def _():
        pltpu.sync_copy(g_v, gains_hbm.at[pl.ds(rbase, _RPW)])

    for j, t in enumerate((t1, t2, t3, t4, t5)):
        stage_v[pl.ds(j * _L, _L)] = t
    pltpu.sync_copy(stage_v, topst.at[pl.ds(s * _K * _L, _K * _L)])

    a0 = zf
    a1 = zf
    a2 = zf
    for j in range(_TR):
        for k in range(128 // _L):
            cg = cntg_v[j, pl.ds(k * _L, _L)]
            a0 = a0 + cg * mk0_v[j, pl.ds(k * _L, _L)]
            a1 = a1 + cg * mk1_v[j, pl.ds(k * _L, _L)]
            a2 = a2 + cg * mk2_v[j, pl.ds(k * _L, _L)]
    sa0 = _lane_sum_splat(a0, lane)
    sa1 = _lane_sum_splat(a1, lane)
    sa2 = _lane_sum_splat(a2, lane)
    svco_v[...] = jnp.where(
        lane == 0, sa0, jnp.where(lane == 1, sa1, jnp.where(lane == 2, sa2, 0.0))
    )
    pltpu.sync_copy(svco_v, svcst.at[pl.ds(s * _L, _L)])
    plsc.subcore_barrier()

    @pl.when((c == 0) & (s == 0))
    def _():
        pltpu.sync_copy(topst, allstage_v)
        pltpu.sync_copy(svcst, allsvc_v)
        g = [allstage_v[pl.ds(j * _L, _L)] for j in range(_K)]
        for t in range(1, _NS):
            for j in range(_K):
                g = _insert_top(g, allstage_v[pl.ds(t * _K * _L + j * _L, _L)])
        widx = _PAD_BASE + lane
        prev = None
        for p in range(_K):
            if prev is None:
                cand = g[0]
            else:
                cand = zi
                for t in g:
                    cand = jnp.maximum(cand, jnp.where(t < prev, t, 0))
            m = _lane_max_splat(cand, lane)
            ok = m >= (1 << 17)
            idx = 0x1FFFF - (m & 0x1FFFF)
            widx = jnp.where((lane == p) & ok, idx, widx)
            prev = m
        widx_v[...] = widx
        wval_v[...] = jnp.full((_L,), _GAIN_UP, jnp.float32)
        pltpu.async_copy(wval_v, gains_hbm.at[widx_v], sem).wait()
        acc = zf
        for t in range(_NS):
            acc = acc + allsvc_v[pl.ds(t * _L, _L)]
        svco_v[...] = acc
        pltpu.sync_copy(svco_v, svc_hbm)


@jax.jit
def kernel(token_sequence, svc_mask, svc_thresholds, svc_decay):
    del svc_thresholds, svc_decay
    tok = token_sequence.astype(jnp.int32).reshape(_SEQ // 128, 128)
    mf = (svc_mask > 0).astype(jnp.float32)
    m0 = mf[:, 0].reshape(_SEQ // 128, 128)
    m1 = mf[:, 1].reshape(_SEQ // 128, 128)
    m2 = mf[:, 2].reshape(_SEQ // 128, 128)
    gains_pad, svc_pad = _spiking_sc_kernel(tok, m0, m1, m2)
    return gains_pad[:_VOCAB], svc_pad[:3]


# speedup vs baseline: 1503.9648x; 1.0035x over previous
import functools

import jax
import jax.numpy as jnp
from jax import lax
from jax.experimental import pallas as pl
from jax.experimental.pallas import tpu as pltpu
from jax.experimental.pallas import tpu_sc as plsc

_VOCAB = 100000
_SEQ = 8192
_L = 16
_NS = 16
_VP = 102400
_RPW = _VP // _NS
_NV = _RPW // _L
_TPW = _SEQ // _NS
_TR = _TPW // 128
_K = 5
_PAD_BASE = _VP - _L
_GAIN_UP = 1.5
_GAIN_DOWN = 0.6

_mesh = plsc.VectorSubcoreMesh(
    core_axis_name="c", subcore_axis_name="s", num_cores=2, num_subcores=_NS
)


def _lane_shuffle(v, perm):
    return v.at[perm].get(mode="promise_in_bounds")


def _lane_max_splat(v, lane):
    for sh in (8, 4, 2, 1):
        v = jnp.maximum(v, _lane_shuffle(v, lane ^ sh))
    return v


def _lane_sum_splat(v, lane):
    for sh in (8, 4, 2, 1):
        v = v + _lane_shuffle(v, lane ^ sh)
    return v


def _insert_top(tops, key):
    out = []
    for j, t in enumerate(tops):
        hi = jnp.maximum(t, key)
        if j + 1 < len(tops):
            key = jnp.minimum(t, key)
        out.append(hi)
    return out


@functools.partial(
    pl.kernel,
    out_type=(
        jax.ShapeDtypeStruct((_VP,), jnp.float32),
        jax.ShapeDtypeStruct((_L,), jnp.float32),
    ),
    mesh=_mesh,
    scratch_types=[
        pltpu.VMEM_SHARED((_VP,), jnp.float32),
        pltpu.VMEM_SHARED((_NS * _K * _L,), jnp.int32),
        pltpu.VMEM_SHARED((_NS * _L,), jnp.float32),
        pltpu.VMEM((_TR, 128), jnp.int32),
        pltpu.VMEM((_TR, 128), jnp.float32),
        pltpu.VMEM((_TR, 128), jnp.float32),
        pltpu.VMEM((_TR, 128), jnp.float32),
        pltpu.VMEM((_TR, 128), jnp.float32),
        pltpu.VMEM((_TR, 128), jnp.float32),
        pltpu.VMEM((_RPW,), jnp.float32),
        pltpu.VMEM((_RPW,), jnp.float32),
        pltpu.VMEM((_K * _L,), jnp.int32),
        pltpu.VMEM((_NS * _K * _L,), jnp.int32),
        pltpu.VMEM((_NS * _L,), jnp.float32),
        pltpu.VMEM((_L,), jnp.int32),
        pltpu.VMEM((_L,), jnp.float32),
        pltpu.VMEM((_L,), jnp.float32),
        pltpu.SemaphoreType.DMA,
    ],
)
def _spiking_sc_kernel(
    tok_hbm, m0_hbm, m1_hbm, m2_hbm, gains_hbm, svc_hbm,
    hist, topst, svcst,
    tok_v, val_v, cntg_v, mk0_v, mk1_v, mk2_v, cnt_v, g_v,
    stage_v, allstage_v, allsvc_v, widx_v, wval_v, svco_v, sem,
):
    c = lax.axis_index("c")
    s = lax.axis_index("s")
    rbase = s * _RPW
    lane = lax.iota(jnp.int32, _L)
    zf = jnp.zeros((_L,), jnp.float32)
    zi = jnp.zeros((_L,), jnp.int32)

    def _zero_body(i, carry):
        for u in range(4):
            cnt_v[pl.ds(pl.multiple_of(i * (4 * _L) + u * _L, _L), _L)] = zf
        return carry

    lax.fori_loop(0, _NV // 4, _zero_body, 0)
    pltpu.sync_copy(cnt_v, hist.at[pl.ds(rbase, _RPW)])
    plsc.subcore_barrier()

    pltpu.sync_copy(tok_hbm.at[pl.ds(s * _TR, _TR)], tok_v)
    ones = jnp.ones((_L,), jnp.float32)
    for j in range(_TR):
        for k in range(128 // _L):
            val_v[j, pl.ds(k * _L, _L)] = ones
    for j in range(_TR):
        pltpu.sync_copy(val_v.at[j], hist.at[tok_v.at[j]], add=True)
    plsc.subcore_barrier()

    pltpu.sync_copy(hist.at[pl.ds(rbase, _RPW)], cnt_v)
    for j in range(_TR):
        pltpu.sync_copy(hist.at[tok_v.at[j]], cntg_v.at[j])
    for msrc, mdst in ((m0_hbm, mk0_v), (m1_hbm, mk1_v), (m2_hbm, mk2_v)):
        pltpu.sync_copy(msrc.at[pl.ds(s * _TR, _TR)], mdst)

    def _scan_body(i, carry):
        t1, t2, t3, t4, t5, kidx = carry
        tops = [t1, t2, t3, t4, t5]
        for u in range(8):
            off = pl.multiple_of(i * (8 * _L) + u * _L, _L)
            cnt = cnt_v[pl.ds(off, _L)]
            g_v[pl.ds(off, _L)] = jnp.where(cnt > 0.0, _GAIN_DOWN, 1.0)
            key = (cnt.astype(jnp.int32) << 17) | (kidx - u * _L)
            tops = _insert_top(tops, key)
        t1, t2, t3, t4, t5 = tops
        return t1, t2, t3, t4, t5, kidx - 8 * _L

    kidx0 = (0x1FFFF - rbase) - lane
    t1, t2, t3, t4, t5, _ = lax.fori_loop(
        0, _NV // 8, _scan_body, (zi, zi, zi, zi, zi, kidx0)
    )

    @pl.when(c == 0)
    def _():
        pltpu.sync_copy(g_v, gains_hbm.at[pl.ds(rbase, _RPW)])

    for j, t in enumerate((t1, t2, t3, t4, t5)):
        stage_v[pl.ds(j * _L, _L)] = t
    pltpu.sync_copy(stage_v, topst.at[pl.ds(s * _K * _L, _K * _L)])

    a0 = zf
    a1 = zf
    a2 = zf
    for j in range(_TR):
        for k in range(128 // _L):
            cg = cntg_v[j, pl.ds(k * _L, _L)]
            a0 = a0 + cg * mk0_v[j, pl.ds(k * _L, _L)]
            a1 = a1 + cg * mk1_v[j, pl.ds(k * _L, _L)]
            a2 = a2 + cg * mk2_v[j, pl.ds(k * _L, _L)]
    sa0 = _lane_sum_splat(a0, lane)
    sa1 = _lane_sum_splat(a1, lane)
    sa2 = _lane_sum_splat(a2, lane)
    svco_v[...] = jnp.where(
        lane == 0, sa0, jnp.where(lane == 1, sa1, jnp.where(lane == 2, sa2, 0.0))
    )
    pltpu.sync_copy(svco_v, svcst.at[pl.ds(s * _L, _L)])
    plsc.subcore_barrier()

    @pl.when((c == 0) & (s == 0))
    def _():
        pltpu.sync_copy(topst, allstage_v)
        pltpu.sync_copy(svcst, allsvc_v)
        g = [allstage_v[pl.ds(j * _L, _L)] for j in range(_K)]
        for t in range(1, _NS):
            for j in range(_K):
                g = _insert_top(g, allstage_v[pl.ds(t * _K * _L + j * _L, _L)])
        widx = _PAD_BASE + lane
        prev = None
        for p in range(_K):
            if prev is None:
                cand = g[0]
            else:
                cand = zi
                for t in g:
                    cand = jnp.maximum(cand, jnp.where(t < prev, t, 0))
            m = _lane_max_splat(cand, lane)
            ok = m >= (1 << 17)
            idx = 0x1FFFF - (m & 0x1FFFF)
            widx = jnp.where((lane == p) & ok, idx, widx)
            prev = m
        widx_v[...] = widx
        wval_v[...] = jnp.full((_L,), _GAIN_UP, jnp.float32)
        pltpu.async_copy(wval_v, gains_hbm.at[widx_v], sem).wait()
        acc = zf
        for t in range(_NS):
            acc = acc + allsvc_v[pl.ds(t * _L, _L)]
        svco_v[...] = acc
        pltpu.sync_copy(svco_v, svc_hbm)


@jax.jit
def kernel(token_sequence, svc_mask, svc_thresholds, svc_decay):
    del svc_thresholds, svc_decay
    tok = token_sequence.astype(jnp.int32).reshape(_SEQ // 128, 128)
    mf = (svc_mask > 0).astype(jnp.float32)
    m0 = mf[:, 0].reshape(_SEQ // 128, 128)
    m1 = mf[:, 1].reshape(_SEQ // 128, 128)
    m2 = mf[:, 2].reshape(_SEQ // 128, 128)
    gains_pad, svc_pad = _spiking_sc_kernel(tok, m0, m1, m2)
    return gains_pad[:_VOCAB], svc_pad[:3]
